# baseline (speedup 1.0000x reference)
import jax
import jax.numpy as jnp
from jax.experimental import pallas as pl
from jax.experimental.pallas import tpu as pltpu

B, T, D = 16, 2048, 1024


def _body(r1_ref, r2_ref, in1, in2, out_ref, sem):
    cps = []
    for b in range(B):
        cps.append(
            pltpu.make_async_copy(
                in1.at[b, pl.ds(r1_ref[b] - 1, 1), :],
                out_ref.at[pl.ds(b, 1), pl.ds(0, D)],
                sem,
            )
        )
        cps.append(
            pltpu.make_async_copy(
                in2.at[b, pl.ds(r2_ref[b] - 1, 1), :],
                out_ref.at[pl.ds(b, 1), pl.ds(D, D)],
                sem,
            )
        )
    for cp in cps:
        cp.start()
    for cp in cps:
        cp.wait()


_grid_spec = pltpu.PrefetchScalarGridSpec(
    num_scalar_prefetch=2,
    grid=(1,),
    in_specs=[
        pl.BlockSpec(memory_space=pl.ANY),
        pl.BlockSpec(memory_space=pl.ANY),
    ],
    out_specs=pl.BlockSpec(memory_space=pl.ANY),
    scratch_shapes=[pltpu.SemaphoreType.DMA],
)

_call = pl.pallas_call(
    _body,
    grid_spec=_grid_spec,
    out_shape=jax.ShapeDtypeStruct((B, 2 * D), jnp.float32),
)


def kernel(output_lstm1, output_lstm2, input_length, support_length):
    return _call(
        input_length.astype(jnp.int32),
        support_length.astype(jnp.int32),
        output_lstm1,
        output_lstm2,
    )


# speedup vs baseline: 2.5544x; 2.5544x over previous
import jax
import jax.numpy as jnp
from jax.experimental import pallas as pl
from jax.experimental.pallas import tpu as pltpu

B, T, D = 16, 2048, 1024


def _body(r1_ref, r2_ref, in1, in2, out_ref, sem):
    cps = []
    for b in range(B):
        cps.append(
            pltpu.make_async_copy(
                in1.at[b, pl.ds(r1_ref[b] - 1, 1), :],
                out_ref.at[pl.ds(b, 1), pl.ds(0, D)],
                sem,
            )
        )
        cps.append(
            pltpu.make_async_copy(
                in2.at[b, pl.ds(r2_ref[b] - 1, 1), :],
                out_ref.at[pl.ds(b, 1), pl.ds(D, D)],
                sem,
            )
        )
    for cp in cps:
        cp.start()
    for cp in cps:
        cp.wait()


_grid_spec = pltpu.PrefetchScalarGridSpec(
    num_scalar_prefetch=2,
    grid=(1,),
    in_specs=[
        pl.BlockSpec(memory_space=pl.ANY),
        pl.BlockSpec(memory_space=pl.ANY),
    ],
    out_specs=pl.BlockSpec((B, 2 * D), lambda i, r1, r2: (0, 0)),
    scratch_shapes=[pltpu.SemaphoreType.DMA],
)

_call = pl.pallas_call(
    _body,
    grid_spec=_grid_spec,
    out_shape=jax.ShapeDtypeStruct((B, 2 * D), jnp.float32),
)


def kernel(output_lstm1, output_lstm2, input_length, support_length):
    return _call(
        input_length.astype(jnp.int32),
        support_length.astype(jnp.int32),
        output_lstm1,
        output_lstm2,
    )
